# Initial kernel scaffold; baseline (speedup 1.0000x reference)
import functools

import jax
import jax.numpy as jnp
from jax import lax
from jax.experimental import pallas as pl
from jax.experimental.pallas import tpu as pltpu
from jax.experimental.pallas import tpu_sc as plsc

_VOCAB = 1000000
_D = 64
_SEQ = 200
_BATCH = 1024
_NC = 2
_NS = 16
_NW = _NC * _NS
_SEQ_PER_W = _BATCH // _NW
_LANES = 16
_VPR = _D // _LANES


def _emb_kernel(x_hbm, tok_hbm, pos_hbm, out_hbm, idx_v, rows_v, pos_v, sem):
    wid = lax.axis_index("s") * _NC + lax.axis_index("c")
    base = wid * (_SEQ_PER_W * _SEQ)

    pltpu.sync_copy(pos_hbm, pos_v)

    def per_seq(i, carry):
        off = base + i * _SEQ
        pltpu.sync_copy(x_hbm.at[pl.ds(off, _SEQ)], idx_v)
        pltpu.async_copy(tok_hbm.at[idx_v], rows_v, sem).wait()

        def add_row(r, c2):
            for c in range(_VPR):
                rows_v[r, pl.ds(c * _LANES, _LANES)] = (
                    rows_v[r, pl.ds(c * _LANES, _LANES)]
                    + pos_v[r, pl.ds(c * _LANES, _LANES)]
                )
            return c2

        lax.fori_loop(0, _SEQ, add_row, 0, unroll=2)
        pltpu.sync_copy(rows_v, out_hbm.at[pl.ds(off, _SEQ)])
        return carry

    lax.fori_loop(0, _SEQ_PER_W, per_seq, 0)


@jax.jit
def _run(x_flat, token_table, position_table):
    mesh = plsc.VectorSubcoreMesh(core_axis_name="c", subcore_axis_name="s")
    f = functools.partial(
        pl.kernel,
        out_type=jax.ShapeDtypeStruct((_BATCH * _SEQ, _D), jnp.float32),
        mesh=mesh,
        scratch_types=[
            pltpu.VMEM((_SEQ,), jnp.int32),
            pltpu.VMEM((_SEQ, _D), jnp.float32),
            pltpu.VMEM((_SEQ, _D), jnp.float32),
            pltpu.SemaphoreType.DMA,
        ],
    )(_emb_kernel)
    return f(x_flat, token_table, position_table)


def kernel(x, token_table, position_table):
    x_flat = x.reshape(-1).astype(jnp.int32)
    out = _run(x_flat, token_table, position_table)
    return out.reshape(_BATCH, _SEQ, _D)


# speedup vs baseline: 1.1470x; 1.1470x over previous
import functools

import jax
import jax.numpy as jnp
from jax import lax
from jax.experimental import pallas as pl
from jax.experimental.pallas import tpu as pltpu
from jax.experimental.pallas import tpu_sc as plsc

_VOCAB = 1000000
_D = 64
_SEQ = 200
_BATCH = 1024
_NC = 2
_NS = 16
_NW = _NC * _NS
_SEQ_PER_W = _BATCH // _NW
_LANES = 16
_VPR = _D // _LANES


def _emb_kernel(x_hbm, tok_hbm, pos_hbm, out_hbm, idx_v, rows_v, pos_v, sem):
    wid = lax.axis_index("s") * _NC + lax.axis_index("c")
    base = wid * (_SEQ_PER_W * _SEQ)

    pltpu.sync_copy(pos_hbm, pos_v)

    def per_seq(i, carry):
        off = base + i * _SEQ
        pltpu.sync_copy(x_hbm.at[pl.ds(off, _SEQ)], idx_v)
        pltpu.async_copy(tok_hbm.at[idx_v], rows_v, sem).wait()

        def add_row(r, c2):
            for c in range(_VPR):
                rows_v[r, pl.ds(c * _LANES, _LANES)] = (
                    rows_v[r, pl.ds(c * _LANES, _LANES)]
                    + pos_v[r, pl.ds(c * _LANES, _LANES)]
                )
            return c2

        lax.fori_loop(0, _SEQ, add_row, 0, unroll=2)
        pltpu.sync_copy(rows_v, out_hbm.at[pl.ds(off, _SEQ)])
        return carry

    lax.fori_loop(0, _SEQ_PER_W, per_seq, 0)


@jax.jit
def _run(x_flat, token_table, position_table):
    mesh = plsc.VectorSubcoreMesh(core_axis_name="c", subcore_axis_name="s")
    f = functools.partial(
        pl.kernel,
        out_type=jax.ShapeDtypeStruct((_BATCH * _SEQ, _D), jnp.float32),
        mesh=mesh,
        scratch_types=[
            pltpu.VMEM((_SEQ,), jnp.int32),
            pltpu.VMEM((_SEQ, _D), jnp.float32),
            pltpu.VMEM((_SEQ, _D), jnp.float32),
            pltpu.SemaphoreType.DMA,
        ],
        compiler_params=pltpu.CompilerParams(use_tc_tiling_on_sc=False),
    )(_emb_kernel)
    return f(x_flat, token_table, position_table)


def kernel(x, token_table, position_table):
    x_flat = x.reshape(-1).astype(jnp.int32)
    out = _run(x_flat, token_table, position_table)
    return out.reshape(_BATCH, _SEQ, _D)


# speedup vs baseline: 1.2096x; 1.0546x over previous
import functools

import jax
import jax.numpy as jnp
from jax import lax
from jax.experimental import pallas as pl
from jax.experimental.pallas import tpu as pltpu
from jax.experimental.pallas import tpu_sc as plsc

_VOCAB = 1000000
_D = 64
_SEQ = 200
_BATCH = 1024
_NC = 2
_NS = 16
_NW = _NC * _NS
_ROWS = _BATCH * _SEQ
_SEQ_PER_CHUNK = 1
_CHUNK = _SEQ * _SEQ_PER_CHUNK
_CHUNKS_PER_W = _ROWS // (_NW * _CHUNK)
_NBUF = 3
_LANES = 16
_VPR = _D // _LANES


def _emb_kernel(x_hbm, tok_hbm, pos_hbm, out_hbm, pos_v, idx, rows, sems):
    sem_g, sem_o = sems
    wid = lax.axis_index("s") * _NC + lax.axis_index("c")
    base = wid * _CHUNKS_PER_W * _CHUNK

    pltpu.sync_copy(pos_hbm, pos_v)

    def stage(c):
        b = c % _NBUF
        off = base + c * _CHUNK
        pltpu.sync_copy(x_hbm.at[pl.ds(off, _CHUNK)], idx[b])
        return pltpu.async_copy(tok_hbm.at[idx[b]], rows[b], sem_g[b])

    def add_positions(b):
        def add_row(r, carry):
            for s in range(_SEQ_PER_CHUNK):
                for v in range(_VPR):
                    rows[b][s * _SEQ + r, pl.ds(v * _LANES, _LANES)] = (
                        rows[b][s * _SEQ + r, pl.ds(v * _LANES, _LANES)]
                        + pos_v[r, pl.ds(v * _LANES, _LANES)]
                    )
            return carry

        lax.fori_loop(0, _SEQ, add_row, 0, unroll=2)

    g = [None] * _CHUNKS_PER_W
    o = [None] * _CHUNKS_PER_W
    g[0] = stage(0)
    for c in range(_CHUNKS_PER_W):
        b = c % _NBUF
        if c + 1 < _CHUNKS_PER_W:
            if c + 1 >= _NBUF:
                o[c + 1 - _NBUF].wait()
            g[c + 1] = stage(c + 1)
        g[c].wait()
        add_positions(b)
        o[c] = pltpu.async_copy(
            rows[b], out_hbm.at[pl.ds(base + c * _CHUNK, _CHUNK)], sem_o[b]
        )
    for c in range(_CHUNKS_PER_W - _NBUF, _CHUNKS_PER_W):
        o[c].wait()


@jax.jit
def _run(x_flat, token_table, position_table):
    mesh = plsc.VectorSubcoreMesh(core_axis_name="c", subcore_axis_name="s")
    f = functools.partial(
        pl.kernel,
        out_type=jax.ShapeDtypeStruct((_ROWS, _D), jnp.float32),
        mesh=mesh,
        scratch_types=[
            pltpu.VMEM((_SEQ, _D), jnp.float32),
            [pltpu.VMEM((_CHUNK,), jnp.int32) for _ in range(_NBUF)],
            [pltpu.VMEM((_CHUNK, _D), jnp.float32) for _ in range(_NBUF)],
            (
                [pltpu.SemaphoreType.DMA for _ in range(_NBUF)],
                [pltpu.SemaphoreType.DMA for _ in range(_NBUF)],
            ),
        ],
        compiler_params=pltpu.CompilerParams(use_tc_tiling_on_sc=False),
    )(_emb_kernel)
    return f(x_flat, token_table, position_table)


def kernel(x, token_table, position_table):
    x_flat = x.reshape(_ROWS).astype(jnp.int32)
    out = _run(x_flat, token_table, position_table)
    return out.reshape(_BATCH, _SEQ, _D)


# speedup vs baseline: 1.2787x; 1.0571x over previous
import functools

import jax
import jax.numpy as jnp
from jax import lax
from jax.experimental import pallas as pl
from jax.experimental.pallas import tpu as pltpu
from jax.experimental.pallas import tpu_sc as plsc

_VOCAB = 1000000
_D = 64
_SEQ = 200
_BATCH = 1024
_NC = 2
_NS = 16
_NW = _NC * _NS
_ROWS = _BATCH * _SEQ
_CHUNK = 2 * _SEQ
_PAIRS = _CHUNK // 2
_CHUNKS_PER_W = _ROWS // (_NW * _CHUNK)
_NBUF = 2
_LANES = 16
_VPR = _D // _LANES


def _emb_kernel(x_hbm, tokp_hbm, pos2_hbm, out_hbm, pos_v, idx, rows, sems):
    sem_g, sem_o = sems
    wid = lax.axis_index("s") * _NC + lax.axis_index("c")
    base = wid * _CHUNKS_PER_W * _CHUNK

    pltpu.sync_copy(pos2_hbm, pos_v)

    def stage(c):
        b = c % _NBUF
        off = base + c * _CHUNK
        pltpu.sync_copy(x_hbm.at[pl.ds(off, _CHUNK)], idx[b])
        return pltpu.async_copy(tokp_hbm.at[idx[b]], rows[b], sem_g[b])

    def compact_add(b):
        def pair(p, s2x100):
            p_abs = s2x100 + p
            for j in range(_VPR):
                rows[b][p_abs, pl.ds(j * _LANES, _LANES)] = (
                    rows[b][2 * p_abs, pl.ds(j * _LANES, _LANES)]
                    + pos_v[p, pl.ds(j * _LANES, _LANES)]
                )
            for j in range(_VPR):
                rows[b][p_abs, pl.ds(_D + j * _LANES, _LANES)] = (
                    rows[b][2 * p_abs + 1, pl.ds(j * _LANES, _LANES)]
                    + pos_v[p, pl.ds(_D + j * _LANES, _LANES)]
                )
            return s2x100

        for s2 in range(2):
            lax.fori_loop(0, _PAIRS // 2, pair, s2 * (_PAIRS // 2), unroll=2)

    g = [None] * _CHUNKS_PER_W
    o = [None] * _CHUNKS_PER_W
    g[0] = stage(0)
    for c in range(_CHUNKS_PER_W):
        b = c % _NBUF
        if c + 1 < _CHUNKS_PER_W:
            if c >= 1:
                o[c - 1].wait()
            g[c + 1] = stage(c + 1)
        g[c].wait()
        compact_add(b)
        o[c] = pltpu.async_copy(
            rows[b].at[pl.ds(0, _PAIRS), :],
            out_hbm.at[pl.ds((base + c * _CHUNK) // 2, _PAIRS)],
            sem_o[b],
        )
    for c in range(_CHUNKS_PER_W - _NBUF, _CHUNKS_PER_W):
        o[c].wait()


@jax.jit
def _run(x_flat, tokp, pos2):
    mesh = plsc.VectorSubcoreMesh(core_axis_name="c", subcore_axis_name="s")
    f = functools.partial(
        pl.kernel,
        out_type=jax.ShapeDtypeStruct((_ROWS // 2, 2 * _D), jnp.float32),
        mesh=mesh,
        scratch_types=[
            pltpu.VMEM((_SEQ // 2, 2 * _D), jnp.float32),
            [pltpu.VMEM((_CHUNK,), jnp.int32) for _ in range(_NBUF)],
            [pltpu.VMEM((_CHUNK, 2 * _D), jnp.float32) for _ in range(_NBUF)],
            (
                [pltpu.SemaphoreType.DMA for _ in range(_NBUF)],
                [pltpu.SemaphoreType.DMA for _ in range(_NBUF)],
            ),
        ],
        compiler_params=pltpu.CompilerParams(use_tc_tiling_on_sc=False),
    )(_emb_kernel)
    return f(x_flat, tokp, pos2)


def kernel(x, token_table, position_table):
    x_flat = x.reshape(_ROWS).astype(jnp.int32)
    tokp = jnp.pad(token_table, ((0, 0), (0, _D)))
    pos2 = position_table.reshape(_SEQ // 2, 2 * _D)
    out = _run(x_flat, tokp, pos2)
    return out.reshape(_BATCH, _SEQ, _D)


# speedup vs baseline: 1.2919x; 1.0103x over previous
import functools

import jax
import jax.numpy as jnp
from jax import lax
from jax.experimental import pallas as pl
from jax.experimental.pallas import tpu as pltpu
from jax.experimental.pallas import tpu_sc as plsc

_VOCAB = 1000000
_D = 64
_SEQ = 200
_BATCH = 1024
_NC = 2
_NS = 16
_NW = _NC * _NS
_ROWS = _BATCH * _SEQ
_CHUNK = 2 * _SEQ
_PAIRS = _CHUNK // 2
_CHUNKS_PER_W = _ROWS // (_NW * _CHUNK)
_NBUF = 2
_LANES = 16
_VPR = _D // _LANES


def _emb_kernel(x_hbm, tokp_hbm, pos2_hbm, out_hbm, pos_v, idx_all, rows, sems):
    sem_g, sem_o = sems
    wid = lax.axis_index("s") * _NC + lax.axis_index("c")
    base = wid * _CHUNKS_PER_W * _CHUNK

    pltpu.sync_copy(pos2_hbm, pos_v)
    pltpu.sync_copy(
        x_hbm.at[pl.ds(base, _CHUNKS_PER_W * _CHUNK)], idx_all
    )

    def stage(c):
        b = c % _NBUF
        return pltpu.async_copy(
            tokp_hbm.at[idx_all.at[pl.ds(c * _CHUNK, _CHUNK)]],
            rows[b],
            sem_g[b],
        )

    def compact_add(b):
        def pair(p, s2x100):
            p_abs = s2x100 + p
            for j in range(_VPR):
                rows[b][p_abs, pl.ds(j * _LANES, _LANES)] = (
                    rows[b][2 * p_abs, pl.ds(j * _LANES, _LANES)]
                    + pos_v[p, pl.ds(j * _LANES, _LANES)]
                )
            for j in range(_VPR):
                rows[b][p_abs, pl.ds(_D + j * _LANES, _LANES)] = (
                    rows[b][2 * p_abs + 1, pl.ds(j * _LANES, _LANES)]
                    + pos_v[p, pl.ds(_D + j * _LANES, _LANES)]
                )
            return s2x100

        for s2 in range(2):
            lax.fori_loop(0, _PAIRS // 2, pair, s2 * (_PAIRS // 2), unroll=2)

    g = [None] * _CHUNKS_PER_W
    o = [None] * _CHUNKS_PER_W
    g[0] = stage(0)
    for c in range(_CHUNKS_PER_W):
        b = c % _NBUF
        if c + 1 < _CHUNKS_PER_W:
            if c >= 1:
                o[c - 1].wait()
            g[c + 1] = stage(c + 1)
        g[c].wait()
        compact_add(b)
        o[c] = pltpu.async_copy(
            rows[b].at[pl.ds(0, _PAIRS), :],
            out_hbm.at[pl.ds((base + c * _CHUNK) // 2, _PAIRS)],
            sem_o[b],
        )
    for c in range(_CHUNKS_PER_W - _NBUF, _CHUNKS_PER_W):
        o[c].wait()


@jax.jit
def _run(x_flat, tokp, pos2):
    mesh = plsc.VectorSubcoreMesh(core_axis_name="c", subcore_axis_name="s")
    f = functools.partial(
        pl.kernel,
        out_type=jax.ShapeDtypeStruct((_ROWS // 2, 2 * _D), jnp.float32),
        mesh=mesh,
        scratch_types=[
            pltpu.VMEM((_SEQ // 2, 2 * _D), jnp.float32),
            pltpu.VMEM((_CHUNKS_PER_W * _CHUNK,), jnp.int32),
            [pltpu.VMEM((_CHUNK, 2 * _D), jnp.float32) for _ in range(_NBUF)],
            (
                [pltpu.SemaphoreType.DMA for _ in range(_NBUF)],
                [pltpu.SemaphoreType.DMA for _ in range(_NBUF)],
            ),
        ],
        compiler_params=pltpu.CompilerParams(use_tc_tiling_on_sc=False),
    )(_emb_kernel)
    return f(x_flat, tokp, pos2)


def kernel(x, token_table, position_table):
    x_flat = x.reshape(_ROWS).astype(jnp.int32)
    tokp = jnp.pad(token_table, ((0, 0), (0, _D)))
    pos2 = position_table.reshape(_SEQ // 2, 2 * _D)
    out = _run(x_flat, tokp, pos2)
    return out.reshape(_BATCH, _SEQ, _D)


# speedup vs baseline: 1.4021x; 1.0854x over previous
import functools

import jax
import jax.numpy as jnp
from jax import lax
from jax.experimental import pallas as pl
from jax.experimental.pallas import tpu as pltpu
from jax.experimental.pallas import tpu_sc as plsc

_VOCAB = 1000000
_D = 64
_SEQ = 200
_BATCH = 1024
_NC = 2
_NS = 16
_NW = _NC * _NS
_ROWS = _BATCH * _SEQ
_CHUNK = 2 * _SEQ
_PAIRS = _CHUNK // 2
_CHUNKS_PER_W = _ROWS // (_NW * _CHUNK)
_NBUF = 2
_LANES = 16
_VPR = _D // _LANES


def _emb_kernel(x_hbm, tokp_hbm, pos2_hbm, out_hbm, pos_v, idx_all, rows, sems):
    sem_g, sem_o = sems
    wid = lax.axis_index("s") * _NC + lax.axis_index("c")
    base = wid * _CHUNKS_PER_W * _CHUNK

    pltpu.sync_copy(pos2_hbm, pos_v)
    pltpu.sync_copy(
        x_hbm.at[pl.ds(base, _CHUNKS_PER_W * _CHUNK)], idx_all
    )

    def stage(c):
        b = c % _NBUF
        return pltpu.async_copy(
            tokp_hbm.at[idx_all.at[pl.ds(c * _CHUNK, _CHUNK)]],
            rows[b],
            sem_g[b],
        )

    def compact_add(b):
        def pair(p, s2x100):
            p_abs = s2x100 + p
            for j in range(_VPR):
                rows[b][p_abs, pl.ds(j * _LANES, _LANES)] = (
                    rows[b][2 * p_abs, pl.ds(j * _LANES, _LANES)]
                    + pos_v[p, pl.ds(j * _LANES, _LANES)]
                )
            for j in range(_VPR):
                rows[b][p_abs, pl.ds(_D + j * _LANES, _LANES)] = (
                    rows[b][2 * p_abs + 1, pl.ds(j * _LANES, _LANES)]
                    + pos_v[p, pl.ds(_D + j * _LANES, _LANES)]
                )
            return s2x100

        for s2 in range(2):
            lax.fori_loop(0, _PAIRS // 2, pair, s2 * (_PAIRS // 2), unroll=2)

    g = [None] * _CHUNKS_PER_W
    o = [None] * _CHUNKS_PER_W
    g[0] = stage(0)
    for c in range(_CHUNKS_PER_W):
        b = c % _NBUF
        if c + 1 < _CHUNKS_PER_W:
            if c >= 1:
                o[c - 1].wait()
            g[c + 1] = stage(c + 1)
        g[c].wait()
        compact_add(b)
        o[c] = pltpu.async_copy(
            rows[b].at[pl.ds(0, _PAIRS), :],
            out_hbm.at[pl.ds((base + c * _CHUNK) // 2, _PAIRS)],
            sem_o[b],
        )
    for c in range(_CHUNKS_PER_W - _NBUF, _CHUNKS_PER_W):
        o[c].wait()


@jax.jit
def _run(x_flat, tokp, pos2):
    mesh = plsc.VectorSubcoreMesh(core_axis_name="c", subcore_axis_name="s")
    f = functools.partial(
        pl.kernel,
        out_type=jax.ShapeDtypeStruct((_ROWS // 2, 2 * _D), jnp.float32),
        mesh=mesh,
        scratch_types=[
            pltpu.VMEM((_SEQ // 2, 2 * _D), jnp.float32),
            pltpu.VMEM((_CHUNKS_PER_W * _CHUNK,), jnp.int32),
            [pltpu.VMEM((_CHUNK, 2 * _D), jnp.float32) for _ in range(_NBUF)],
            (
                [pltpu.SemaphoreType.DMA for _ in range(_NBUF)],
                [pltpu.SemaphoreType.DMA for _ in range(_NBUF)],
            ),
        ],
        compiler_params=pltpu.CompilerParams(use_tc_tiling_on_sc=False),
    )(_emb_kernel)
    return f(x_flat, tokp, pos2)


_TB = 2048


def _transpose_body(in_ref, out_ref):
    t = jnp.transpose(in_ref[...])
    out_ref[...] = jnp.concatenate(
        [t, jnp.zeros((_TB, _D), jnp.float32)], axis=1
    )


@jax.jit
def _relayout(tokT):
    grid = (_VOCAB + _TB - 1) // _TB
    return pl.pallas_call(
        _transpose_body,
        grid=(grid,),
        in_specs=[pl.BlockSpec((_D, _TB), lambda g: (0, g))],
        out_specs=pl.BlockSpec((_TB, 2 * _D), lambda g: (g, 0)),
        out_shape=jax.ShapeDtypeStruct((_VOCAB, 2 * _D), jnp.float32),
    )(tokT)


def kernel(x, token_table, position_table):
    x_flat = x.reshape(_ROWS).astype(jnp.int32)
    tokp = _relayout(token_table.T)
    pos2 = position_table.reshape(_SEQ // 2, 2 * _D)
    out = _run(x_flat, tokp, pos2)
    return out.reshape(_BATCH, _SEQ, _D)
